# baseline (speedup 1.0000x reference)
import functools

import jax
import jax.numpy as jnp
from jax import lax
from jax.experimental import pallas as pl
from jax.experimental.pallas import tpu as pltpu

_BE = 640
_BEW = 648
_BM = 1000


def _elu(v):
    return jnp.where(v > 0, v, jnp.exp(jnp.minimum(v, 0.0)) - 1.0)


def _lrelu(v):
    return jnp.where(v >= 0, v, 0.2 * v)


def _mm_kernel(apply_elu, h_ref, wl_ref, wr_ref, xl_ref, xr_ref):
    hb = h_ref[...]
    if apply_elu:
        hb = _elu(hb)
    xl_ref[...] = jnp.dot(hb, wl_ref[...], preferred_element_type=jnp.float32)
    xr_ref[...] = jnp.dot(hb, wr_ref[...], preferred_element_type=jnp.float32)


def _proj(h, Wl, Wr, apply_elu):
    n, a = h.shape
    b = Wl.shape[1]
    return pl.pallas_call(
        functools.partial(_mm_kernel, apply_elu),
        grid=(n // _BM,),
        in_specs=[
            pl.BlockSpec((_BM, a), lambda i: (i, 0)),
            pl.BlockSpec((a, b), lambda i: (0, 0)),
            pl.BlockSpec((a, b), lambda i: (0, 0)),
        ],
        out_specs=[
            pl.BlockSpec((_BM, b), lambda i: (i, 0)),
            pl.BlockSpec((_BM, b), lambda i: (i, 0)),
        ],
        out_shape=[
            jax.ShapeDtypeStruct((n, b), jnp.float32),
            jax.ShapeDtypeStruct((n, b), jnp.float32),
        ],
    )(h, Wl, Wr)


def _edge_e_kernel(xs_ref, xd_ref, att_ref, e_ref):
    m = _lrelu(xs_ref[...] + xd_ref[...])
    e_ref[0] = jnp.sum(m * att_ref[...], axis=1, keepdims=True)


def _edge_e(xs_g, xd_g, att):
    e, b = xs_g.shape
    c = e // _BE
    return pl.pallas_call(
        _edge_e_kernel,
        grid=(c,),
        in_specs=[
            pl.BlockSpec((_BE, b), lambda k: (k, 0)),
            pl.BlockSpec((_BE, b), lambda k: (k, 0)),
            pl.BlockSpec((1, b), lambda k: (0, 0)),
        ],
        out_specs=pl.BlockSpec((1, _BE, 1), lambda k: (k, 0, 0)),
        out_shape=jax.ShapeDtypeStruct((c, _BE, 1), jnp.float32),
    )(xs_g, xd_g, att)


def _agg_kernel(lo_ref, xs_ref, ex_ref, den_ref, r_ref, comp_ref, alpha_ref):
    k = pl.program_id(0)

    @pl.when(k == 0)
    def _():
        comp_ref[...] = jnp.zeros_like(comp_ref)

    alpha = ex_ref[0] / (den_ref[0] + 1e-16)
    alpha_ref[0] = alpha
    msgs = xs_ref[...] * alpha
    lo = lo_ref[k] * 8
    local = r_ref[0] - lo
    onehot = (local == lax.broadcasted_iota(jnp.int32, (_BE, _BEW), 1)
              ).astype(jnp.float32)
    contrib = lax.dot_general(onehot, msgs, (((0,), (0,)), ((), ())),
                              preferred_element_type=jnp.float32)
    comp_ref[pl.ds(lo, _BEW), :] += contrib


def _aggregate(xs_g, n, ex3, den3, r3, lo):
    e, b = xs_g.shape
    c = e // _BE
    rows = ((n + _BEW + 7) // 8) * 8
    grid_spec = pltpu.PrefetchScalarGridSpec(
        num_scalar_prefetch=1,
        grid=(c,),
        in_specs=[
            pl.BlockSpec((_BE, b), lambda k, s: (k, 0)),
            pl.BlockSpec((1, _BE, 1), lambda k, s: (k, 0, 0)),
            pl.BlockSpec((1, _BE, 1), lambda k, s: (k, 0, 0)),
            pl.BlockSpec((1, _BE, 1), lambda k, s: (k, 0, 0)),
        ],
        out_specs=[
            pl.BlockSpec((rows, b), lambda k, s: (0, 0)),
            pl.BlockSpec((1, _BE, 1), lambda k, s: (k, 0, 0)),
        ],
    )
    return pl.pallas_call(
        _agg_kernel,
        grid_spec=grid_spec,
        out_shape=[
            jax.ShapeDtypeStruct((rows, b), jnp.float32),
            jax.ShapeDtypeStruct((c, _BE, 1), jnp.float32),
        ],
    )(lo, xs_g, ex3, den3, r3)


def _pad_cols(w, b):
    return jnp.pad(w, ((0, 0), (0, b - w.shape[1])))


def _pad_rows(w, a):
    return jnp.pad(w, ((0, a - w.shape[0]), (0, 0)))


def kernel(x, edge_index, Wl0, Wr0, att0, Wl1, Wr1, att1, Wl2, Wr2, att2,
           Wl3, Wr3, att3, Wl4, Wr4, att4, Wl5, Wr5, att5):
    N = x.shape[0]
    E = edge_index.shape[1]
    c = E // _BE

    params = [
        (Wl0, Wr0, att0),
        (Wl1, Wr1, att1),
        (_pad_cols(Wl2, 128), _pad_cols(Wr2, 128), jnp.pad(att2, (0, 80))),
        (_pad_rows(Wl3, 128), _pad_rows(Wr3, 128), att3),
        (Wl4, Wr4, att4),
        (Wl5, Wr5, att5),
    ]

    src = edge_index[0]
    dst = edge_index[1]
    perm = jnp.argsort(dst)
    src_s = src[perm]
    dst_s = dst[perm]

    newseg = jnp.concatenate(
        [jnp.ones((1,), jnp.int32),
         (dst_s[1:] != dst_s[:-1]).astype(jnp.int32)])
    r = jnp.cumsum(newseg) - 1
    lo = (r.reshape(c, _BE)[:, 0] // 8).astype(jnp.int32)
    r3 = r.reshape(c, _BE, 1).astype(jnp.int32)
    u = jnp.zeros((N,), jnp.int32).at[r].set(dst_s)

    H = x
    Henc = None
    alpha_enc = None
    for i in range(6):
        Wl, Wr, att = params[i]
        b = Wl.shape[1]
        xl, xr = _proj(H, Wl, Wr, i in (1, 2, 4, 5))
        xs_g = xl[src_s]
        xd_g = xr[dst_s]
        e = _edge_e(xs_g, xd_g, att.reshape(1, b)).reshape(E)
        emax = jax.ops.segment_max(e, dst_s, num_segments=N)
        ex = jnp.exp(e - emax[dst_s])
        den = jax.ops.segment_sum(ex, dst_s, num_segments=N)
        comp, alpha3 = _aggregate(
            xs_g, N,
            ex.reshape(c, _BE, 1),
            den[dst_s].reshape(c, _BE, 1),
            r3, lo)
        H = jnp.zeros((N, b), jnp.float32).at[u].add(comp[:N])
        if i == 2:
            Henc = H[:, :48]
            alpha_enc = jnp.zeros((E,), jnp.float32).at[perm].set(
                alpha3.reshape(E))
    return (Henc, H, alpha_enc)
